# baseline (speedup 1.0000x reference)
import functools

import jax
import jax.numpy as jnp
from jax import lax
from jax.experimental import pallas as pl
from jax.experimental.pallas import tpu as pltpu
from jax.experimental.pallas import tpu_sc as plsc

NC, NS, NL = 2, 16, 16
N_NODES = 40000
N_PAD = 40960
NBKT = 10
BKT = 4096
TPB = BKT // NS
E_CHUNK = 2000
FIRE = 128
KB = 3
KBAT = KB * FIRE
KCAP = 8192
MB = 3
MBAT = MB * FIRE
MCAP = 2048
D = 128
ROWS_BLK = 2000

_MESH = functools.partial(
    plsc.VectorSubcoreMesh, core_axis_name="c", subcore_axis_name="s",
    num_cores=NC, num_subcores=NS)


def _k1_body(x_hbm, src_hbm, dst_hbm, agg_hbm, deg_hbm,
             sp_agg, sp_deg, zbuf, zdeg1d, iden, dstc, srcc,
             pend_src, pend_ldst, rows, degpart, sem, sem2):
    c = lax.axis_index("c")
    s = lax.axis_index("s")
    E = src_hbm.shape[0]
    epw = E // NS
    zeros16 = jnp.zeros((NL,), jnp.float32)
    ones16 = jnp.ones((NL,), jnp.float32)
    iota16 = lax.iota(jnp.int32, NL)
    zi16 = jnp.zeros((NL,), jnp.int32)

    for r in range(zbuf.shape[0]):
        for cc in range(D // NL):
            zbuf[r, pl.ds(cc * NL, NL)] = zeros16
    zdeg1d[pl.ds(0, NL)] = zeros16
    for r in range(TPB // NL):
        zdeg1d[pl.ds(r * NL, NL)] = zeros16
    for r in range(iden.shape[0]):
        for cc in range(D // NL):
            iden[r, pl.ds(cc * NL, NL)] = iota16 + (r * D + cc * NL)

    def process(cnt):
        for r in range(KBAT // NL):
            pos = cnt + r * NL + iota16
            plsc.store_scatter(pend_src, [pos], zi16)
            plsc.store_scatter(pend_ldst, [pos >> 7, pos & (FIRE - 1)],
                               zi16 + BKT)
        nbat = (cnt + KBAT - 1) // KBAT

        def batch(b, _):
            descs = []
            for q in range(KB):
                base = b * KBAT + q * FIRE
                descs.append(pltpu.async_copy(
                    x_hbm.at[pend_src.at[pl.ds(base, FIRE)]],
                    rows.at[pl.ds(q * FIRE, FIRE)], sem))
            for d in descs:
                d.wait()
            descs2 = []
            for q in range(KB):
                descs2.append(pltpu.async_copy(
                    rows.at[pl.ds(q * FIRE, FIRE)],
                    sp_agg.at[pend_ldst.at[b * KB + q]], sem2, add=True))
            for d in descs2:
                d.wait()
            return 0

        lax.fori_loop(0, nbat, batch, 0)
        return jnp.int32(0)

    for k in range(NBKT // NC):
        bid = k * NC + c
        base = bid * BKT

        for kk in range(TPB // zbuf.shape[0]):
            pltpu.sync_copy(zbuf, sp_agg.at[pl.ds(s * TPB + kk * zbuf.shape[0],
                                                  zbuf.shape[0])])
        pltpu.sync_copy(zdeg1d, sp_deg.at[pl.ds(s * TPB, TPB)])

        def zdeg(i, _):
            degpart[pl.ds(i * NL, NL)] = zeros16
            return 0
        lax.fori_loop(0, BKT // NL, zdeg, 0)
        plsc.subcore_barrier()

        def chunk_body(ic, cnt):
            off = s * epw + ic * E_CHUNK
            pltpu.sync_copy(dst_hbm.at[pl.ds(off, E_CHUNK)], dstc)
            pltpu.sync_copy(src_hbm.at[pl.ds(off, E_CHUNK)], srcc)

            def vec_body(i, cnt):
                dv = dstc[pl.ds(i * NL, NL)]
                sv = srcc[pl.ds(i * NL, NL)]
                ldv = dv - base
                mv = (dv >= base) & (dv < base + BKT)
                plsc.addupdate_scatter(degpart, [ldv], ones16, mask=mv)
                pos = cnt + plsc.cumsum(mv.astype(jnp.int32)) - 1
                plsc.store_scatter(pend_src, [pos], sv, mask=mv)
                plsc.store_scatter(pend_ldst, [pos >> 7, pos & (FIRE - 1)],
                                   ldv, mask=mv)
                return cnt + jnp.sum(mv.astype(jnp.int32))

            cnt = lax.fori_loop(0, E_CHUNK // NL, vec_body, cnt)
            return lax.cond(cnt >= KCAP - E_CHUNK, process, lambda n: n, cnt)

        cnt = lax.fori_loop(0, epw // E_CHUNK, chunk_body, jnp.int32(0))
        cnt = process(cnt)

        descs = []
        for r in range(iden.shape[0]):
            descs.append(pltpu.async_copy(
                degpart.at[pl.ds(r * FIRE, FIRE)],
                sp_deg.at[iden.at[r]], sem2, add=True))
        for d in descs:
            d.wait()
        plsc.subcore_barrier()

        pltpu.sync_copy(sp_agg.at[pl.ds(s * TPB, TPB)],
                        agg_hbm.at[pl.ds(base + s * TPB, TPB)])
        pltpu.sync_copy(sp_deg.at[pl.ds(s * TPB, TPB)],
                        deg_hbm.at[pl.ds(base + s * TPB, TPB)])
        plsc.subcore_barrier()


def _k1(x, srcs, dsts):
    return pl.kernel(
        _k1_body,
        out_type=(jax.ShapeDtypeStruct((N_PAD, D), jnp.float32),
                  jax.ShapeDtypeStruct((N_PAD,), jnp.float32)),
        mesh=_MESH(),
        compiler_params=pltpu.CompilerParams(needs_layout_passes=False),
        scratch_types=[
            pltpu.VMEM_SHARED((BKT + 1, D), jnp.float32),
            pltpu.VMEM_SHARED((BKT + NL,), jnp.float32),
            pltpu.VMEM((16, D), jnp.float32),
            pltpu.VMEM((TPB,), jnp.float32),
            pltpu.VMEM((BKT // FIRE, FIRE), jnp.int32),
            pltpu.VMEM((E_CHUNK,), jnp.int32),
            pltpu.VMEM((E_CHUNK,), jnp.int32),
            pltpu.VMEM((KCAP + KBAT,), jnp.int32),
            pltpu.VMEM(((KCAP + KBAT) // FIRE, FIRE), jnp.int32),
            pltpu.VMEM((KBAT, D), jnp.float32),
            pltpu.VMEM((BKT,), jnp.float32),
            pltpu.SemaphoreType.DMA,
            pltpu.SemaphoreType.DMA,
        ],
    )(x, srcs, dsts)


M_POOL = 10240
M_MB = 10304
RANGE = 320
EPW = 40000
EPW2 = EPW + FIRE
TRASH = RANGE


def _k3_body(src_hbm, dst_hbm, mask_hbm, pj_hbm, ps_hbm, cnts_hbm, jj_hbm,
             inv, maskb, jjbuf, dstc, srcc, pend_j, pend_s,
             fire_j, fire_s, cstage):
    c = lax.axis_index("c")
    s = lax.axis_index("s")
    wid = s * NC + c
    iota16 = lax.iota(jnp.int32, NL)
    neg16 = jnp.full((NL,), -1, jnp.int32)

    pltpu.sync_copy(mask_hbm, maskb)

    def zinv(i, _):
        inv[pl.ds(i * NL, NL)] = neg16
        return 0
    lax.fori_loop(0, N_PAD // NL, zinv, 0)

    def binv(i, _):
        nodes = maskb[pl.ds(i * NL, NL)]
        plsc.store_scatter(inv, [nodes], iota16 + i * NL)
        return 0
    lax.fori_loop(0, M_POOL // NL, binv, 0)

    def do_fire(carry):
        cnt, nf = carry
        for r in range(FIRE // NL):
            fire_j[pl.ds(r * NL, NL)] = pend_j[pl.ds(r * NL, NL)]
            fire_s[pl.ds(r * NL, NL)] = pend_s[pl.ds(r * NL, NL)]
        pend_j[pl.ds(0, NL)] = pend_j[pl.ds(FIRE, NL)]
        pend_s[pl.ds(0, NL)] = pend_s[pl.ds(FIRE, NL)]
        pltpu.sync_copy(fire_j, pj_hbm.at[pl.ds(wid * EPW2 + nf * FIRE, FIRE)])
        pltpu.sync_copy(fire_s, ps_hbm.at[pl.ds(wid * EPW2 + nf * FIRE, FIRE)])
        return cnt - FIRE, nf + 1

    def chunk_body(ic, carry):
        off = wid * EPW + ic * E_CHUNK
        pltpu.sync_copy(dst_hbm.at[pl.ds(off, E_CHUNK)], dstc)
        pltpu.sync_copy(src_hbm.at[pl.ds(off, E_CHUNK)], srcc)

        def vec_body(i, carry):
            cnt, nf = carry
            dv = dstc[pl.ds(i * NL, NL)]
            sv = srcc[pl.ds(i * NL, NL)]
            jv = plsc.load_gather(inv, [dv])
            mv = jv >= 0
            pos = cnt + plsc.cumsum(mv.astype(jnp.int32)) - 1
            plsc.store_scatter(pend_j, [pos], jv, mask=mv)
            plsc.store_scatter(pend_s, [pos], sv, mask=mv)
            cnt = cnt + jnp.sum(mv.astype(jnp.int32))
            return lax.cond(cnt >= FIRE, do_fire, lambda cr: cr, (cnt, nf))

        return lax.fori_loop(0, E_CHUNK // NL, vec_body, carry)

    cnt, nf = lax.fori_loop(0, EPW // E_CHUNK, chunk_body,
                            (jnp.int32(0), jnp.int32(0)))
    pltpu.sync_copy(pend_j.at[pl.ds(0, FIRE)],
                    pj_hbm.at[pl.ds(wid * EPW2 + nf * FIRE, FIRE)])
    pltpu.sync_copy(pend_s.at[pl.ds(0, FIRE)],
                    ps_hbm.at[pl.ds(wid * EPW2 + nf * FIRE, FIRE)])
    cstage[pl.ds(0, NL)] = jnp.zeros((NL,), jnp.int32) + (nf * FIRE + cnt)
    pltpu.sync_copy(cstage, cnts_hbm.at[wid])

    @pl.when((c == 0) & (s == 0))
    def _():
        def bjj(i, _):
            nodes = maskb[pl.ds(i * NL, NL)]
            jjbuf[pl.ds(i * NL, NL)] = plsc.load_gather(inv, [nodes])
            return 0
        lax.fori_loop(0, M_MB // NL, bjj, 0)
        pltpu.sync_copy(jjbuf, jj_hbm)


def _k3(srcs, dsts, mask_pad):
    return pl.kernel(
        _k3_body,
        out_type=(jax.ShapeDtypeStruct((NC * NS * EPW2,), jnp.int32),
                  jax.ShapeDtypeStruct((NC * NS * EPW2,), jnp.int32),
                  jax.ShapeDtypeStruct((NC * NS, NL), jnp.int32),
                  jax.ShapeDtypeStruct((M_MB,), jnp.int32)),
        mesh=_MESH(),
        compiler_params=pltpu.CompilerParams(needs_layout_passes=False),
        scratch_types=[
            pltpu.VMEM((N_PAD,), jnp.int32),
            pltpu.VMEM((M_MB,), jnp.int32),
            pltpu.VMEM((M_MB,), jnp.int32),
            pltpu.VMEM((E_CHUNK,), jnp.int32),
            pltpu.VMEM((E_CHUNK,), jnp.int32),
            pltpu.VMEM((FIRE + NL,), jnp.int32),
            pltpu.VMEM((FIRE + NL,), jnp.int32),
            pltpu.VMEM((FIRE,), jnp.int32),
            pltpu.VMEM((FIRE,), jnp.int32),
            pltpu.VMEM((NL,), jnp.int32),
        ],
    )(srcs, dsts, mask_pad)


def _k4_body(h_hbm, pj_hbm, ps_hbm, cnts_hbm, mask_hbm, pooled_hbm,
             maskb, acc, rows, cj, cs, pend_l, pend_s, idx128, cntsb, sem):
    c = lax.axis_index("c")
    s = lax.axis_index("s")
    wid = s * NC + c
    j0 = wid * RANGE
    iota16 = lax.iota(jnp.int32, NL)
    zi16 = jnp.zeros((NL,), jnp.int32)

    pltpu.sync_copy(mask_hbm, maskb)
    pltpu.sync_copy(cnts_hbm, cntsb)

    for k in range(3):
        def fill(i, _):
            idx128[pl.ds(i * NL, NL)] = maskb[pl.ds(j0 + k * FIRE + i * NL, NL)]
            return 0
        lax.fori_loop(0, FIRE // NL, fill, 0)
        pltpu.async_copy(h_hbm.at[idx128], acc.at[pl.ds(k * FIRE, FIRE)],
                         sem).wait()

    def process(cnt):
        for r in range(MBAT // NL):
            pos = cnt + r * NL + iota16
            plsc.store_scatter(pend_s, [pos], zi16)
            plsc.store_scatter(pend_l, [pos], zi16 + TRASH)
        nbat = (cnt + MBAT - 1) // MBAT

        def batch(b, _):
            descs = []
            for q in range(MB):
                base = b * MBAT + q * FIRE
                descs.append(pltpu.async_copy(
                    h_hbm.at[pend_s.at[pl.ds(base, FIRE)]],
                    rows.at[pl.ds(q * FIRE, FIRE)], sem))
            for q in range(MB):
                descs[q].wait()
                def rmw(g, _):
                    lvec = pend_l[pl.ds(b * MBAT + q * FIRE + g * NL, NL)]
                    for kk in range(NL):
                        ld = lvec[kk]
                        e = q * FIRE + g * NL + kk
                        for cc in range(D // NL):
                            sl = pl.ds(cc * NL, NL)
                            acc[ld, sl] = jnp.maximum(acc[ld, sl],
                                                      rows[e, sl])
                    return 0
                lax.fori_loop(0, FIRE // NL, rmw, 0)
            return 0

        lax.fori_loop(0, nbat, batch, 0)
        return jnp.int32(0)

    def tile_body(t, _):
        cnt_t = cntsb[t, pl.ds(0, NL)][0]
        nchunks = (cnt_t + E_CHUNK - 1) // E_CHUNK

        def chunk_body(ic, cnt):
            off = t * EPW2 + ic * E_CHUNK
            pltpu.sync_copy(pj_hbm.at[pl.ds(off, E_CHUNK)], cj)
            pltpu.sync_copy(ps_hbm.at[pl.ds(off, E_CHUNK)], cs)

            def vec_body(i, cnt):
                gidx = ic * E_CHUNK + i * NL + iota16
                jv = cj[pl.ds(i * NL, NL)]
                sv = cs[pl.ds(i * NL, NL)]
                mv = (gidx < cnt_t) & (jv >= j0) & (jv < j0 + RANGE)
                pos = cnt + plsc.cumsum(mv.astype(jnp.int32)) - 1
                plsc.store_scatter(pend_l, [pos], jv - j0, mask=mv)
                plsc.store_scatter(pend_s, [pos], sv, mask=mv)
                return cnt + jnp.sum(mv.astype(jnp.int32))

            cnt = lax.fori_loop(0, E_CHUNK // NL, vec_body, cnt)
            return lax.cond(cnt >= MCAP - E_CHUNK, process, lambda n: n, cnt)

        cnt = lax.fori_loop(0, nchunks, chunk_body, jnp.int32(0))
        return process(cnt)

    lax.fori_loop(0, NC * NS, tile_body, jnp.int32(0))
    pltpu.sync_copy(acc.at[pl.ds(0, RANGE)], pooled_hbm.at[pl.ds(j0, RANGE)])


def _k4(h, pairs_j, pairs_src, counts, mask_pad):
    return pl.kernel(
        _k4_body,
        out_type=jax.ShapeDtypeStruct((M_POOL, D), jnp.float32),
        mesh=_MESH(),
        compiler_params=pltpu.CompilerParams(needs_layout_passes=False),
        scratch_types=[
            pltpu.VMEM((M_MB,), jnp.int32),
            pltpu.VMEM((3 * FIRE, D), jnp.float32),
            pltpu.VMEM((MBAT, D), jnp.float32),
            pltpu.VMEM((E_CHUNK,), jnp.int32),
            pltpu.VMEM((E_CHUNK,), jnp.int32),
            pltpu.VMEM((MCAP + MBAT,), jnp.int32),
            pltpu.VMEM((MCAP + MBAT,), jnp.int32),
            pltpu.VMEM((FIRE,), jnp.int32),
            pltpu.VMEM((NC * NS, NL), jnp.int32),
            pltpu.SemaphoreType.DMA,
        ],
    )(h, pairs_j, pairs_src, counts, mask_pad)


def _k5_body(pooled_hbm, jj_hbm, out_hbm, jjb, rows, sem):
    c = lax.axis_index("c")
    s = lax.axis_index("s")
    wid = s * NC + c
    j0 = wid * RANGE
    for k in range(3):
        pltpu.sync_copy(jj_hbm.at[pl.ds(j0 + k * FIRE, FIRE)], jjb)
        pltpu.async_copy(pooled_hbm.at[jjb],
                         rows.at[pl.ds(k * FIRE, FIRE)], sem).wait()
    n_out = jnp.maximum(jnp.minimum(10000 - j0, RANGE), 0)

    def wr(k, _):
        pltpu.sync_copy(rows.at[pl.ds(k * 40, 40)],
                        out_hbm.at[pl.ds(j0 + k * 40, 40)])
        return 0
    lax.fori_loop(0, n_out // 40, wr, 0)


def _k5(pooled, jj):
    return pl.kernel(
        _k5_body,
        out_type=jax.ShapeDtypeStruct((10000, D), jnp.float32),
        mesh=_MESH(),
        compiler_params=pltpu.CompilerParams(needs_layout_passes=False),
        scratch_types=[
            pltpu.VMEM((FIRE,), jnp.int32),
            pltpu.VMEM((3 * FIRE, D), jnp.float32),
            pltpu.SemaphoreType.DMA,
        ],
    )(pooled, jj)


def _dense_body(x_ref, agg_ref, deg_ref, wr_ref, wn_ref, b_ref, h_ref):
    x = x_ref[...]
    agg = agg_ref[...] / jnp.maximum(deg_ref[...], 1.0)
    h = (
        jnp.dot(x, wr_ref[...], preferred_element_type=jnp.float32,
                precision=jax.lax.Precision.HIGHEST)
        + jnp.dot(agg, wn_ref[...], preferred_element_type=jnp.float32,
                  precision=jax.lax.Precision.HIGHEST)
        + b_ref[...]
    )
    h_ref[...] = jnp.where(h > 0, h, jnp.exp(jnp.minimum(h, 0.0)) - 1.0)


def _dense_stage(x, agg, deg, W_root, W_nbr, b):
    n = x.shape[0]
    d = x.shape[1]
    grid = (n // ROWS_BLK,)
    return pl.pallas_call(
        _dense_body,
        grid=grid,
        in_specs=[
            pl.BlockSpec((ROWS_BLK, d), lambda i: (i, 0)),
            pl.BlockSpec((ROWS_BLK, d), lambda i: (i, 0)),
            pl.BlockSpec((ROWS_BLK, 1), lambda i: (i, 0)),
            pl.BlockSpec((d, d), lambda i: (0, 0)),
            pl.BlockSpec((d, d), lambda i: (0, 0)),
            pl.BlockSpec((d,), lambda i: (0,)),
        ],
        out_specs=pl.BlockSpec((ROWS_BLK, d), lambda i: (i, 0)),
        out_shape=jax.ShapeDtypeStruct((n, d), jnp.float32),
    )(x, agg, deg, W_root, W_nbr, b)


def kernel(x, edge_index, pooling_mask, W_root, W_nbr, b):
    N = x.shape[0]
    src, dst = edge_index[0], edge_index[1]
    mask_pad = jnp.pad(pooling_mask.astype(jnp.int32), (0, M_MB - 10000))
    agg_p, deg_p = _k1(x, src, dst)
    agg = agg_p[:N]
    deg = deg_p[:N, None]
    h = _dense_stage(x, agg, deg, W_root, W_nbr, b)
    pairs_j, pairs_src, counts, jj = _k3(src, dst, mask_pad)
    pooled = _k4(h, pairs_j, pairs_src, counts, mask_pad)
    return _k5(pooled, jj)


# speedup vs baseline: 2.7455x; 2.7455x over previous
import functools

import jax
import jax.numpy as jnp
from jax import lax
from jax.experimental import pallas as pl
from jax.experimental.pallas import tpu as pltpu
from jax.experimental.pallas import tpu_sc as plsc

NC, NS, NL = 2, 16, 16
N_NODES = 40000
N_PAD = 40960
NBKT = 10
BKT = 4096
TPB = BKT // NS
E_CHUNK = 2000
FIRE = 128
KB = 3
KBAT = KB * FIRE
KCAP = 8192
MB = 3
MBAT = MB * FIRE
MCAP = 4096
D = 128
ROWS_BLK = 2000

_MESH = functools.partial(
    plsc.VectorSubcoreMesh, core_axis_name="c", subcore_axis_name="s",
    num_cores=NC, num_subcores=NS)


def _k1_body(x_hbm, src_hbm, dst_hbm, agg_hbm, deg_hbm,
             sp_agg, sp_deg, zbuf, zdeg1d, iden, dstc, srcc,
             pend_src, pend_ldst, rows, degpart, sem, sem2):
    c = lax.axis_index("c")
    s = lax.axis_index("s")
    E = src_hbm.shape[0]
    epw = E // NS
    zeros16 = jnp.zeros((NL,), jnp.float32)
    ones16 = jnp.ones((NL,), jnp.float32)
    iota16 = lax.iota(jnp.int32, NL)
    zi16 = jnp.zeros((NL,), jnp.int32)

    for r in range(zbuf.shape[0]):
        for cc in range(D // NL):
            zbuf[r, pl.ds(cc * NL, NL)] = zeros16
    zdeg1d[pl.ds(0, NL)] = zeros16
    for r in range(TPB // NL):
        zdeg1d[pl.ds(r * NL, NL)] = zeros16
    for r in range(iden.shape[0]):
        for cc in range(D // NL):
            iden[r, pl.ds(cc * NL, NL)] = iota16 + (r * D + cc * NL)

    def process(cnt):
        for r in range(KBAT // NL):
            pos = cnt + r * NL + iota16
            plsc.store_scatter(pend_src, [pos], zi16)
            plsc.store_scatter(pend_ldst, [pos >> 7, pos & (FIRE - 1)],
                               zi16 + BKT)
        nbat = (cnt + KBAT - 1) // KBAT

        def batch(b, _):
            descs = []
            for q in range(KB):
                base = b * KBAT + q * FIRE
                descs.append(pltpu.async_copy(
                    x_hbm.at[pend_src.at[pl.ds(base, FIRE)]],
                    rows.at[pl.ds(q * FIRE, FIRE)], sem))
            for d in descs:
                d.wait()
            descs2 = []
            for q in range(KB):
                descs2.append(pltpu.async_copy(
                    rows.at[pl.ds(q * FIRE, FIRE)],
                    sp_agg.at[pend_ldst.at[b * KB + q]], sem2, add=True))
            for d in descs2:
                d.wait()
            return 0

        lax.fori_loop(0, nbat, batch, 0)
        return jnp.int32(0)

    for k in range(NBKT // NC):
        bid = k * NC + c
        base = bid * BKT

        for kk in range(TPB // zbuf.shape[0]):
            pltpu.sync_copy(zbuf, sp_agg.at[pl.ds(s * TPB + kk * zbuf.shape[0],
                                                  zbuf.shape[0])])
        pltpu.sync_copy(zdeg1d, sp_deg.at[pl.ds(s * TPB, TPB)])

        def zdeg(i, _):
            degpart[pl.ds(i * NL, NL)] = zeros16
            return 0
        lax.fori_loop(0, BKT // NL, zdeg, 0)
        plsc.subcore_barrier()

        def chunk_body(ic, cnt):
            off = s * epw + ic * E_CHUNK
            pltpu.sync_copy(dst_hbm.at[pl.ds(off, E_CHUNK)], dstc)
            pltpu.sync_copy(src_hbm.at[pl.ds(off, E_CHUNK)], srcc)

            def vec_body(i, cnt):
                dv = dstc[pl.ds(i * NL, NL)]
                sv = srcc[pl.ds(i * NL, NL)]
                ldv = dv - base
                mv = (dv >= base) & (dv < base + BKT)
                plsc.addupdate_scatter(degpart, [ldv], ones16, mask=mv)
                pos = cnt + plsc.cumsum(mv.astype(jnp.int32)) - 1
                plsc.store_scatter(pend_src, [pos], sv, mask=mv)
                plsc.store_scatter(pend_ldst, [pos >> 7, pos & (FIRE - 1)],
                                   ldv, mask=mv)
                return cnt + jnp.sum(mv.astype(jnp.int32))

            cnt = lax.fori_loop(0, E_CHUNK // NL, vec_body, cnt)
            return lax.cond(cnt >= KCAP - E_CHUNK, process, lambda n: n, cnt)

        cnt = lax.fori_loop(0, epw // E_CHUNK, chunk_body, jnp.int32(0))
        cnt = process(cnt)

        descs = []
        for r in range(iden.shape[0]):
            descs.append(pltpu.async_copy(
                degpart.at[pl.ds(r * FIRE, FIRE)],
                sp_deg.at[iden.at[r]], sem2, add=True))
        for d in descs:
            d.wait()
        plsc.subcore_barrier()

        pltpu.sync_copy(sp_agg.at[pl.ds(s * TPB, TPB)],
                        agg_hbm.at[pl.ds(base + s * TPB, TPB)])
        pltpu.sync_copy(sp_deg.at[pl.ds(s * TPB, TPB)],
                        deg_hbm.at[pl.ds(base + s * TPB, TPB)])
        plsc.subcore_barrier()


def _k1(x, srcs, dsts):
    return pl.kernel(
        _k1_body,
        out_type=(jax.ShapeDtypeStruct((N_PAD, D), jnp.float32),
                  jax.ShapeDtypeStruct((N_PAD,), jnp.float32)),
        mesh=_MESH(),
        compiler_params=pltpu.CompilerParams(needs_layout_passes=False),
        scratch_types=[
            pltpu.VMEM_SHARED((BKT + 1, D), jnp.float32),
            pltpu.VMEM_SHARED((BKT + NL,), jnp.float32),
            pltpu.VMEM((16, D), jnp.float32),
            pltpu.VMEM((TPB,), jnp.float32),
            pltpu.VMEM((BKT // FIRE, FIRE), jnp.int32),
            pltpu.VMEM((E_CHUNK,), jnp.int32),
            pltpu.VMEM((E_CHUNK,), jnp.int32),
            pltpu.VMEM((KCAP + KBAT,), jnp.int32),
            pltpu.VMEM(((KCAP + KBAT) // FIRE, FIRE), jnp.int32),
            pltpu.VMEM((KBAT, D), jnp.float32),
            pltpu.VMEM((BKT,), jnp.float32),
            pltpu.SemaphoreType.DMA,
            pltpu.SemaphoreType.DMA,
        ],
    )(x, srcs, dsts)


M_POOL = 10240
M_MB = 10304
RANGE = 320
EPW = 40000
EPW2 = EPW + FIRE
TRASH = RANGE


def _k3_body(src_hbm, dst_hbm, mask_hbm, pj_hbm, ps_hbm, cnts_hbm, jj_hbm,
             inv, maskb, jjbuf, dstc, srcc, pend_j, pend_s,
             fire_j, fire_s, cstage):
    c = lax.axis_index("c")
    s = lax.axis_index("s")
    wid = s * NC + c
    iota16 = lax.iota(jnp.int32, NL)
    neg16 = jnp.full((NL,), -1, jnp.int32)

    pltpu.sync_copy(mask_hbm, maskb)

    def zinv(i, _):
        inv[pl.ds(i * NL, NL)] = neg16
        return 0
    lax.fori_loop(0, N_PAD // NL, zinv, 0)

    def binv(i, _):
        nodes = maskb[pl.ds(i * NL, NL)]
        plsc.store_scatter(inv, [nodes], iota16 + i * NL)
        return 0
    lax.fori_loop(0, M_POOL // NL, binv, 0)

    def do_fire(carry):
        cnt, nf = carry
        for r in range(FIRE // NL):
            fire_j[pl.ds(r * NL, NL)] = pend_j[pl.ds(r * NL, NL)]
            fire_s[pl.ds(r * NL, NL)] = pend_s[pl.ds(r * NL, NL)]
        pend_j[pl.ds(0, NL)] = pend_j[pl.ds(FIRE, NL)]
        pend_s[pl.ds(0, NL)] = pend_s[pl.ds(FIRE, NL)]
        pltpu.sync_copy(fire_j, pj_hbm.at[pl.ds(wid * EPW2 + nf * FIRE, FIRE)])
        pltpu.sync_copy(fire_s, ps_hbm.at[pl.ds(wid * EPW2 + nf * FIRE, FIRE)])
        return cnt - FIRE, nf + 1

    def chunk_body(ic, carry):
        off = wid * EPW + ic * E_CHUNK
        pltpu.sync_copy(dst_hbm.at[pl.ds(off, E_CHUNK)], dstc)
        pltpu.sync_copy(src_hbm.at[pl.ds(off, E_CHUNK)], srcc)

        def vec_body(i, carry):
            cnt, nf = carry
            dv = dstc[pl.ds(i * NL, NL)]
            sv = srcc[pl.ds(i * NL, NL)]
            jv = plsc.load_gather(inv, [dv])
            mv = jv >= 0
            pos = cnt + plsc.cumsum(mv.astype(jnp.int32)) - 1
            plsc.store_scatter(pend_j, [pos], jv, mask=mv)
            plsc.store_scatter(pend_s, [pos], sv, mask=mv)
            cnt = cnt + jnp.sum(mv.astype(jnp.int32))
            return lax.cond(cnt >= FIRE, do_fire, lambda cr: cr, (cnt, nf))

        return lax.fori_loop(0, E_CHUNK // NL, vec_body, carry)

    cnt, nf = lax.fori_loop(0, EPW // E_CHUNK, chunk_body,
                            (jnp.int32(0), jnp.int32(0)))
    pltpu.sync_copy(pend_j.at[pl.ds(0, FIRE)],
                    pj_hbm.at[pl.ds(wid * EPW2 + nf * FIRE, FIRE)])
    pltpu.sync_copy(pend_s.at[pl.ds(0, FIRE)],
                    ps_hbm.at[pl.ds(wid * EPW2 + nf * FIRE, FIRE)])
    cstage[pl.ds(0, NL)] = jnp.zeros((NL,), jnp.int32) + (nf * FIRE + cnt)
    pltpu.sync_copy(cstage, cnts_hbm.at[wid])

    @pl.when((c == 0) & (s == 0))
    def _():
        def bjj(i, _):
            nodes = maskb[pl.ds(i * NL, NL)]
            jjbuf[pl.ds(i * NL, NL)] = plsc.load_gather(inv, [nodes])
            return 0
        lax.fori_loop(0, M_MB // NL, bjj, 0)
        pltpu.sync_copy(jjbuf, jj_hbm)


def _k3(srcs, dsts, mask_pad):
    return pl.kernel(
        _k3_body,
        out_type=(jax.ShapeDtypeStruct((NC * NS * EPW2,), jnp.int32),
                  jax.ShapeDtypeStruct((NC * NS * EPW2,), jnp.int32),
                  jax.ShapeDtypeStruct((NC * NS, NL), jnp.int32),
                  jax.ShapeDtypeStruct((M_MB,), jnp.int32)),
        mesh=_MESH(),
        compiler_params=pltpu.CompilerParams(needs_layout_passes=False),
        scratch_types=[
            pltpu.VMEM((N_PAD,), jnp.int32),
            pltpu.VMEM((M_MB,), jnp.int32),
            pltpu.VMEM((M_MB,), jnp.int32),
            pltpu.VMEM((E_CHUNK,), jnp.int32),
            pltpu.VMEM((E_CHUNK,), jnp.int32),
            pltpu.VMEM((FIRE + NL,), jnp.int32),
            pltpu.VMEM((FIRE + NL,), jnp.int32),
            pltpu.VMEM((FIRE,), jnp.int32),
            pltpu.VMEM((FIRE,), jnp.int32),
            pltpu.VMEM((NL,), jnp.int32),
        ],
    )(srcs, dsts, mask_pad)


def _k4_body(h_hbm, pj_hbm, ps_hbm, cnts_hbm, mask_hbm, pooled_hbm,
             maskb, acc, rows, cj, cs, pend_l, pend_s, idx128, cntsb, sem):
    c = lax.axis_index("c")
    s = lax.axis_index("s")
    wid = s * NC + c
    j0 = wid * RANGE
    iota16 = lax.iota(jnp.int32, NL)
    zi16 = jnp.zeros((NL,), jnp.int32)

    pltpu.sync_copy(mask_hbm, maskb)
    pltpu.sync_copy(cnts_hbm, cntsb)

    for k in range(3):
        def fill(i, _):
            idx128[pl.ds(i * NL, NL)] = maskb[pl.ds(j0 + k * FIRE + i * NL, NL)]
            return 0
        lax.fori_loop(0, FIRE // NL, fill, 0)
        pltpu.async_copy(h_hbm.at[idx128], acc.at[pl.ds(k * FIRE, FIRE)],
                         sem).wait()

    def process(cnt):
        for r in range(MBAT // NL):
            pos = cnt + r * NL + iota16
            plsc.store_scatter(pend_s, [pos], zi16)
            plsc.store_scatter(pend_l, [pos], zi16 + TRASH)
        nbat = (cnt + MBAT - 1) // MBAT

        def batch(b, _):
            descs = []
            for q in range(MB):
                base = b * MBAT + q * FIRE
                descs.append(pltpu.async_copy(
                    h_hbm.at[pend_s.at[pl.ds(base, FIRE)]],
                    rows.at[pl.ds(q * FIRE, FIRE)], sem))
            for q in range(MB):
                descs[q].wait()
                def rmw(g, _):
                    lvec = pend_l[pl.ds(b * MBAT + q * FIRE + g * NL, NL)]
                    for kk in range(NL):
                        ld = lvec[kk]
                        e = q * FIRE + g * NL + kk
                        for cc in range(D // NL):
                            sl = pl.ds(cc * NL, NL)
                            acc[ld, sl] = jnp.maximum(acc[ld, sl],
                                                      rows[e, sl])
                    return 0
                lax.fori_loop(0, FIRE // NL, rmw, 0)
            return 0

        lax.fori_loop(0, nbat, batch, 0)
        return jnp.int32(0)

    def tile_body(t, cnt):
        cnt_t = cntsb[t, pl.ds(0, NL)][0]
        nchunks = (cnt_t + E_CHUNK - 1) // E_CHUNK

        def chunk_body(ic, cnt):
            off = t * EPW2 + ic * E_CHUNK
            pltpu.sync_copy(pj_hbm.at[pl.ds(off, E_CHUNK)], cj)
            pltpu.sync_copy(ps_hbm.at[pl.ds(off, E_CHUNK)], cs)

            def vec_body(i, cnt):
                gidx = ic * E_CHUNK + i * NL + iota16
                jv = cj[pl.ds(i * NL, NL)]
                sv = cs[pl.ds(i * NL, NL)]
                mv = (gidx < cnt_t) & (jv >= j0) & (jv < j0 + RANGE)
                pos = cnt + plsc.cumsum(mv.astype(jnp.int32)) - 1
                plsc.store_scatter(pend_l, [pos], jv - j0, mask=mv)
                plsc.store_scatter(pend_s, [pos], sv, mask=mv)
                return cnt + jnp.sum(mv.astype(jnp.int32))

            cnt = lax.fori_loop(0, E_CHUNK // NL, vec_body, cnt)
            return lax.cond(cnt >= MCAP - E_CHUNK, process, lambda n: n, cnt)

        return lax.fori_loop(0, nchunks, chunk_body, cnt)

    cnt = lax.fori_loop(0, NC * NS, tile_body, jnp.int32(0))
    process(cnt)
    pltpu.sync_copy(acc.at[pl.ds(0, RANGE)], pooled_hbm.at[pl.ds(j0, RANGE)])


def _k4(h, pairs_j, pairs_src, counts, mask_pad):
    return pl.kernel(
        _k4_body,
        out_type=jax.ShapeDtypeStruct((M_POOL, D), jnp.float32),
        mesh=_MESH(),
        compiler_params=pltpu.CompilerParams(needs_layout_passes=False),
        scratch_types=[
            pltpu.VMEM((M_MB,), jnp.int32),
            pltpu.VMEM((3 * FIRE, D), jnp.float32),
            pltpu.VMEM((MBAT, D), jnp.float32),
            pltpu.VMEM((E_CHUNK,), jnp.int32),
            pltpu.VMEM((E_CHUNK,), jnp.int32),
            pltpu.VMEM((MCAP + MBAT,), jnp.int32),
            pltpu.VMEM((MCAP + MBAT,), jnp.int32),
            pltpu.VMEM((FIRE,), jnp.int32),
            pltpu.VMEM((NC * NS, NL), jnp.int32),
            pltpu.SemaphoreType.DMA,
        ],
    )(h, pairs_j, pairs_src, counts, mask_pad)


def _k5_body(pooled_hbm, jj_hbm, out_hbm, jjb, rows, sem):
    c = lax.axis_index("c")
    s = lax.axis_index("s")
    wid = s * NC + c
    j0 = wid * RANGE
    for k in range(3):
        pltpu.sync_copy(jj_hbm.at[pl.ds(j0 + k * FIRE, FIRE)], jjb)
        pltpu.async_copy(pooled_hbm.at[jjb],
                         rows.at[pl.ds(k * FIRE, FIRE)], sem).wait()
    n_out = jnp.maximum(jnp.minimum(10000 - j0, RANGE), 0)

    def wr(k, _):
        pltpu.sync_copy(rows.at[pl.ds(k * 40, 40)],
                        out_hbm.at[pl.ds(j0 + k * 40, 40)])
        return 0
    lax.fori_loop(0, n_out // 40, wr, 0)


def _k5(pooled, jj):
    return pl.kernel(
        _k5_body,
        out_type=jax.ShapeDtypeStruct((10000, D), jnp.float32),
        mesh=_MESH(),
        compiler_params=pltpu.CompilerParams(needs_layout_passes=False),
        scratch_types=[
            pltpu.VMEM((FIRE,), jnp.int32),
            pltpu.VMEM((3 * FIRE, D), jnp.float32),
            pltpu.SemaphoreType.DMA,
        ],
    )(pooled, jj)


def _dense_body(x_ref, agg_ref, deg_ref, wr_ref, wn_ref, b_ref, h_ref):
    x = x_ref[...]
    agg = agg_ref[...] / jnp.maximum(deg_ref[...], 1.0)
    h = (
        jnp.dot(x, wr_ref[...], preferred_element_type=jnp.float32,
                precision=jax.lax.Precision.HIGHEST)
        + jnp.dot(agg, wn_ref[...], preferred_element_type=jnp.float32,
                  precision=jax.lax.Precision.HIGHEST)
        + b_ref[...]
    )
    h_ref[...] = jnp.where(h > 0, h, jnp.exp(jnp.minimum(h, 0.0)) - 1.0)


def _dense_stage(x, agg, deg, W_root, W_nbr, b):
    n = x.shape[0]
    d = x.shape[1]
    grid = (n // ROWS_BLK,)
    return pl.pallas_call(
        _dense_body,
        grid=grid,
        in_specs=[
            pl.BlockSpec((ROWS_BLK, d), lambda i: (i, 0)),
            pl.BlockSpec((ROWS_BLK, d), lambda i: (i, 0)),
            pl.BlockSpec((ROWS_BLK, 1), lambda i: (i, 0)),
            pl.BlockSpec((d, d), lambda i: (0, 0)),
            pl.BlockSpec((d, d), lambda i: (0, 0)),
            pl.BlockSpec((d,), lambda i: (0,)),
        ],
        out_specs=pl.BlockSpec((ROWS_BLK, d), lambda i: (i, 0)),
        out_shape=jax.ShapeDtypeStruct((n, d), jnp.float32),
    )(x, agg, deg, W_root, W_nbr, b)


def kernel(x, edge_index, pooling_mask, W_root, W_nbr, b):
    N = x.shape[0]
    src, dst = edge_index[0], edge_index[1]
    mask_pad = jnp.pad(pooling_mask.astype(jnp.int32), (0, M_MB - 10000))
    agg_p, deg_p = _k1(x, src, dst)
    agg = agg_p[:N]
    deg = deg_p[:N, None]
    h = _dense_stage(x, agg, deg, W_root, W_nbr, b)
    pairs_j, pairs_src, counts, jj = _k3(src, dst, mask_pad)
    pooled = _k4(h, pairs_j, pairs_src, counts, mask_pad)
    return _k5(pooled, jj)
